# baseline (speedup 1.0000x reference)
import functools

import jax
import jax.numpy as jnp
from jax import lax
from jax.experimental import pallas as pl
from jax.experimental.pallas import tpu as pltpu
from jax.experimental.pallas import tpu_sc as plsc

NC, NS, L = 2, 16, 16
NW = NC * NS
BATCH, HIST, D = 16384, 200, 64
RPW = BATCH // NW
TB = 16
NBLK = RPW // TB
DEPTH = 8
G0 = 128
G1 = HIST - G0


def _sc_body(tok_hbm, table_hbm, out_hbm, tokbuf, idxbuf, rows, outbuf, tsem,
             gsem0, gsem1, gsem2, gsem3, gsem4, gsem5, gsem6, gsem7):
    wid = lax.axis_index("s") * NC + lax.axis_index("c")
    base = wid * RPW

    def issue_gathers(r_local, tslot, rslot, gsem):
        rl = r_local & (TB - 1)
        for i in range(12):
            v = tokbuf[tslot, rl, pl.ds(i * L, L)]
            idxbuf[rslot, pl.ds(i * L, L)] = v & 4095
        v = tokbuf[tslot, rl, pl.ds(HIST - L, L)]
        idxbuf[rslot, pl.ds(HIST - L, L)] = v & 4095
        pltpu.async_copy(
            table_hbm.at[idxbuf.at[rslot, pl.ds(0, G0)]],
            rows.at[rslot, pl.ds(0, G0)], gsem)
        pltpu.async_copy(
            table_hbm.at[idxbuf.at[rslot, pl.ds(G0, G1)]],
            rows.at[rslot, pl.ds(G0, G1)], gsem)

    def wait_gathers(rslot, gsem):
        pltpu.make_async_copy(
            table_hbm.at[tokbuf.at[0, 0, pl.ds(0, G0)]],
            rows.at[rslot, pl.ds(0, G0)], gsem).wait()
        pltpu.make_async_copy(
            table_hbm.at[tokbuf.at[0, 0, pl.ds(G0, G1)]],
            rows.at[rslot, pl.ds(G0, G1)], gsem).wait()

    def compute_row(r_local, tslot, rslot):
        a0 = rows[rslot, 0, pl.ds(0, L)]
        outbuf[r_local & (TB - 1), pl.ds(0, L)] = a0
        outbuf[r_local & (TB - 1), pl.ds(L, L)] = a0
        outbuf[r_local & (TB - 1), pl.ds(2 * L, L)] = a0
        outbuf[r_local & (TB - 1), pl.ds(3 * L, L)] = a0

    gsems = (gsem0, gsem1, gsem2, gsem3, gsem4, gsem5, gsem6, gsem7)

    pltpu.sync_copy(tok_hbm.at[pl.ds(base, TB)], tokbuf.at[0])
    for i in range(DEPTH - 1):
        issue_gathers(i, 0, i, gsems[i])

    GPB = TB // DEPTH
    NG = RPW // DEPTH

    def group(g, _):
        b = g // GPB
        tslot = b & 1
        r = DEPTH * g

        @pl.when(jnp.logical_and((g & (GPB - 1)) == 0, b + 1 < NBLK))
        def _():
            pltpu.async_copy(
                tok_hbm.at[pl.ds(base + (b + 1) * TB, TB)],
                tokbuf.at[(b + 1) & 1], tsem)

        @pl.when(jnp.logical_and((g & (GPB - 1)) == GPB - 1, b + 1 < NBLK))
        def _():
            pltpu.make_async_copy(
                tok_hbm.at[pl.ds(base, TB)], tokbuf.at[0], tsem).wait()

        for s in range(DEPTH):
            rr = r + s
            nxt = rr + DEPTH - 1
            nslot = (s + DEPTH - 1) % DEPTH

            wait_gathers(s, gsems[s])

            if s == 0:
                issue_gathers(nxt, (nxt // TB) & 1, nslot, gsems[nslot])
            else:
                @pl.when(g < NG - 1)
                def _():
                    issue_gathers(nxt, (nxt // TB) & 1, nslot, gsems[nslot])

            compute_row(rr, tslot, s)
        return ()

    lax.fori_loop(0, NG, group, ())
    pltpu.sync_copy(outbuf, out_hbm.at[pl.ds(base, TB)])


_sc_kernel = functools.partial(
    pl.kernel,
    out_type=jax.ShapeDtypeStruct((BATCH, D), jnp.float32),
    mesh=plsc.VectorSubcoreMesh(
        core_axis_name="c", subcore_axis_name="s",
        num_cores=NC, num_subcores=NS),
    scratch_types=[
        pltpu.VMEM((2, TB, HIST), jnp.int32),
        pltpu.VMEM((DEPTH, HIST), jnp.int32),
        pltpu.VMEM((DEPTH, HIST, D), jnp.float32),
        pltpu.VMEM((TB, D), jnp.float32),
        pltpu.SemaphoreType.DMA,
        pltpu.SemaphoreType.DMA,
        pltpu.SemaphoreType.DMA,
        pltpu.SemaphoreType.DMA,
        pltpu.SemaphoreType.DMA,
        pltpu.SemaphoreType.DMA,
        pltpu.SemaphoreType.DMA,
        pltpu.SemaphoreType.DMA,
        pltpu.SemaphoreType.DMA,
    ],
    compiler_params=pltpu.CompilerParams(
        needs_layout_passes=False, use_tc_tiling_on_sc=False),
)(_sc_body)


def kernel(token_ids, table):
    return _sc_kernel(token_ids, table)


# speedup vs baseline: 1.1280x; 1.1280x over previous
import functools

import jax
import jax.numpy as jnp
from jax import lax
from jax.experimental import pallas as pl
from jax.experimental.pallas import tpu as pltpu
from jax.experimental.pallas import tpu_sc as plsc

NC, NS, L = 2, 16, 16
NW = NC * NS
BATCH, HIST, D = 16384, 200, 64
RPW = BATCH // NW
TB = 16
NBLK = RPW // TB
DEPTH = 8
G0 = 128
G1 = HIST - G0
FULL_CHUNKS = HIST // L
TAIL = HIST - FULL_CHUNKS * L


def _sc_body(tok_hbm, table_hbm, out_hbm, tokbuf, rows, outbuf, tsem, osem,
             gsem0, gsem1, gsem2, gsem3, gsem4, gsem5, gsem6, gsem7):
    wid = lax.axis_index("s") * NC + lax.axis_index("c")
    base = wid * RPW

    def issue_gathers(r_local, tslot, rslot, gsem):
        rl = r_local & (TB - 1)
        pltpu.async_copy(
            table_hbm.at[tokbuf.at[tslot, rl, pl.ds(0, G0)]],
            rows.at[rslot, pl.ds(0, G0)], gsem)
        pltpu.async_copy(
            table_hbm.at[tokbuf.at[tslot, rl, pl.ds(G0, G1)]],
            rows.at[rslot, pl.ds(G0, G1)], gsem)

    def wait_gathers(rslot, gsem):
        pltpu.make_async_copy(
            table_hbm.at[tokbuf.at[0, 0, pl.ds(0, G0)]],
            rows.at[rslot, pl.ds(0, G0)], gsem).wait()
        pltpu.make_async_copy(
            table_hbm.at[tokbuf.at[0, 0, pl.ds(G0, G1)]],
            rows.at[rslot, pl.ds(G0, G1)], gsem).wait()

    def compute_row(r_local, tslot, rslot, oslot):
        rl = r_local & (TB - 1)
        zeros = jnp.zeros((L,), jnp.float32)
        izeros = jnp.zeros((L,), jnp.int32)

        def chunk(i, carry):
            a0, a1, a2, a3, cnt = carry
            t = tokbuf[tslot, rl, pl.ds(i * L, L)]
            cnt = cnt + plsc.all_reduce_population_count(t != 0)
            for k in range(L):
                rr = i * L + k
                a0 = a0 + rows[rslot, rr, pl.ds(0, L)]
                a1 = a1 + rows[rslot, rr, pl.ds(L, L)]
                a2 = a2 + rows[rslot, rr, pl.ds(2 * L, L)]
                a3 = a3 + rows[rslot, rr, pl.ds(3 * L, L)]
            return (a0, a1, a2, a3, cnt)

        a0, a1, a2, a3, cnt = lax.fori_loop(
            0, FULL_CHUNKS, chunk, (zeros, zeros, zeros, zeros, izeros))

        t = tokbuf[tslot, rl, pl.ds(HIST - L, L)]
        lane = lax.iota(jnp.int32, 16)
        cnt = cnt + plsc.all_reduce_population_count(
            (lane >= L - TAIL) & (t != 0))
        for k in range(TAIL):
            rr = FULL_CHUNKS * L + k
            a0 = a0 + rows[rslot, rr, pl.ds(0, L)]
            a1 = a1 + rows[rslot, rr, pl.ds(L, L)]
            a2 = a2 + rows[rslot, rr, pl.ds(2 * L, L)]
            a3 = a3 + rows[rslot, rr, pl.ds(3 * L, L)]

        sv = 1.0 / jnp.maximum(cnt.astype(jnp.float32), 1.0)
        outbuf[oslot, rl, pl.ds(0, L)] = a0 * sv
        outbuf[oslot, rl, pl.ds(L, L)] = a1 * sv
        outbuf[oslot, rl, pl.ds(2 * L, L)] = a2 * sv
        outbuf[oslot, rl, pl.ds(3 * L, L)] = a3 * sv

    gsems = (gsem0, gsem1, gsem2, gsem3, gsem4, gsem5, gsem6, gsem7)

    pltpu.sync_copy(tok_hbm.at[pl.ds(base, TB)], tokbuf.at[0])
    for i in range(DEPTH - 1):
        issue_gathers(i, 0, i, gsems[i])

    GPB = TB // DEPTH
    NG = RPW // DEPTH

    def group(g, _):
        b = g // GPB
        tslot = b & 1
        oslot = b & 1
        r = DEPTH * g
        first_in_block = (g & (GPB - 1)) == 0
        last_in_block = (g & (GPB - 1)) == GPB - 1

        @pl.when(jnp.logical_and(first_in_block, b + 1 < NBLK))
        def _():
            pltpu.async_copy(
                tok_hbm.at[pl.ds(base + (b + 1) * TB, TB)],
                tokbuf.at[(b + 1) & 1], tsem)

        @pl.when(jnp.logical_and(first_in_block, b >= 2))
        def _():
            pltpu.make_async_copy(
                outbuf.at[0], out_hbm.at[pl.ds(base, TB)], osem).wait()

        @pl.when(jnp.logical_and(last_in_block, b + 1 < NBLK))
        def _():
            pltpu.make_async_copy(
                tok_hbm.at[pl.ds(base, TB)], tokbuf.at[0], tsem).wait()

        for s in range(DEPTH):
            rr = r + s
            nxt = rr + DEPTH - 1
            nslot = (s + DEPTH - 1) % DEPTH

            wait_gathers(s, gsems[s])

            if s == 0:
                issue_gathers(nxt, (nxt // TB) & 1, nslot, gsems[nslot])
            else:
                @pl.when(g < NG - 1)
                def _():
                    issue_gathers(nxt, (nxt // TB) & 1, nslot, gsems[nslot])

            compute_row(rr, tslot, s, oslot)

        @pl.when(last_in_block)
        def _():
            pltpu.async_copy(
                outbuf.at[oslot], out_hbm.at[pl.ds(base + b * TB, TB)], osem)
        return ()

    lax.fori_loop(0, NG, group, ())
    pltpu.make_async_copy(
        outbuf.at[0], out_hbm.at[pl.ds(base, TB)], osem).wait()
    pltpu.make_async_copy(
        outbuf.at[0], out_hbm.at[pl.ds(base, TB)], osem).wait()


_sc_kernel = functools.partial(
    pl.kernel,
    out_type=jax.ShapeDtypeStruct((BATCH, D), jnp.float32),
    mesh=plsc.VectorSubcoreMesh(
        core_axis_name="c", subcore_axis_name="s",
        num_cores=NC, num_subcores=NS),
    scratch_types=[
        pltpu.VMEM((2, TB, HIST), jnp.int32),
        pltpu.VMEM((DEPTH, HIST, D), jnp.float32),
        pltpu.VMEM((2, TB, D), jnp.float32),
        pltpu.SemaphoreType.DMA,
        pltpu.SemaphoreType.DMA,
        pltpu.SemaphoreType.DMA,
        pltpu.SemaphoreType.DMA,
        pltpu.SemaphoreType.DMA,
        pltpu.SemaphoreType.DMA,
        pltpu.SemaphoreType.DMA,
        pltpu.SemaphoreType.DMA,
        pltpu.SemaphoreType.DMA,
        pltpu.SemaphoreType.DMA,
    ],
    compiler_params=pltpu.CompilerParams(
        needs_layout_passes=False, use_tc_tiling_on_sc=False),
)(_sc_body)


def kernel(token_ids, table):
    return _sc_kernel(token_ids, table)


# speedup vs baseline: 1.1741x; 1.0408x over previous
import functools

import jax
import jax.numpy as jnp
from jax import lax
from jax.experimental import pallas as pl
from jax.experimental.pallas import tpu as pltpu
from jax.experimental.pallas import tpu_sc as plsc

NC, NS, L = 2, 16, 16
NW = NC * NS
BATCH, HIST, D = 16384, 200, 64
RPW = BATCH // NW
TB = 16
NBLK = RPW // TB
DEPTH = 4
G0 = 128
G1 = HIST - G0
FULL_CHUNKS = HIST // L
TAIL = HIST - FULL_CHUNKS * L


def _sc_body(tok_hbm, table_hbm, out_hbm, tokbuf, rows, outbuf,
             tsem, gsem0, gsem1, gsem2, gsem3):
    wid = lax.axis_index("s") * NC + lax.axis_index("c")
    base = wid * RPW

    def issue_gathers(r_local, tslot, rslot, gsem):
        rl = r_local & (TB - 1)
        pltpu.async_copy(
            table_hbm.at[tokbuf.at[tslot, rl, pl.ds(0, G0)]],
            rows.at[rslot, pl.ds(0, G0)], gsem)
        pltpu.async_copy(
            table_hbm.at[tokbuf.at[tslot, rl, pl.ds(G0, G1)]],
            rows.at[rslot, pl.ds(G0, G1)], gsem)

    def wait_gathers(rslot, gsem):
        pltpu.make_async_copy(
            table_hbm.at[tokbuf.at[0, 0, pl.ds(0, G0)]],
            rows.at[rslot, pl.ds(0, G0)], gsem).wait()
        pltpu.make_async_copy(
            table_hbm.at[tokbuf.at[0, 0, pl.ds(G0, G1)]],
            rows.at[rslot, pl.ds(G0, G1)], gsem).wait()

    def compute_row(r_local, tslot, rslot):
        rl = r_local & (TB - 1)
        zeros = jnp.zeros((L,), jnp.float32)
        izeros = jnp.zeros((L,), jnp.int32)

        def chunk(i, carry):
            a0, a1, a2, a3, cnt = carry
            t = tokbuf[tslot, rl, pl.ds(i * L, L)]
            cnt = cnt + plsc.all_reduce_population_count(t != 0)
            for k in range(L):
                rr = i * L + k
                a0 = a0 + rows[rslot, rr, pl.ds(0, L)]
                a1 = a1 + rows[rslot, rr, pl.ds(L, L)]
                a2 = a2 + rows[rslot, rr, pl.ds(2 * L, L)]
                a3 = a3 + rows[rslot, rr, pl.ds(3 * L, L)]
            return (a0, a1, a2, a3, cnt)

        a0, a1, a2, a3, cnt = lax.fori_loop(
            0, FULL_CHUNKS, chunk, (zeros, zeros, zeros, zeros, izeros))

        t = tokbuf[tslot, rl, pl.ds(HIST - L, L)]
        lane = lax.iota(jnp.int32, 16)
        cnt = cnt + plsc.all_reduce_population_count(
            (lane >= L - TAIL) & (t != 0))
        for k in range(TAIL):
            rr = FULL_CHUNKS * L + k
            a0 = a0 + rows[rslot, rr, pl.ds(0, L)]
            a1 = a1 + rows[rslot, rr, pl.ds(L, L)]
            a2 = a2 + rows[rslot, rr, pl.ds(2 * L, L)]
            a3 = a3 + rows[rslot, rr, pl.ds(3 * L, L)]

        sv = 1.0 / jnp.maximum(cnt.astype(jnp.float32), 1.0)
        outbuf[r_local, pl.ds(0, L)] = a0 * sv
        outbuf[r_local, pl.ds(L, L)] = a1 * sv
        outbuf[r_local, pl.ds(2 * L, L)] = a2 * sv
        outbuf[r_local, pl.ds(3 * L, L)] = a3 * sv

    gsems = (gsem0, gsem1, gsem2, gsem3)

    pltpu.sync_copy(tok_hbm.at[pl.ds(base, TB)], tokbuf.at[0])
    for i in range(DEPTH - 1):
        issue_gathers(i, 0, i, gsems[i])

    GPB = TB // DEPTH
    NG = RPW // DEPTH

    def group(g, _):
        b = g // GPB
        tslot = b & 1
        r = DEPTH * g

        @pl.when(jnp.logical_and((g & (GPB - 1)) == 0, b + 1 < NBLK))
        def _():
            pltpu.async_copy(
                tok_hbm.at[pl.ds(base + (b + 1) * TB, TB)],
                tokbuf.at[(b + 1) & 1], tsem)

        @pl.when(jnp.logical_and((g & (GPB - 1)) == GPB - 1, b + 1 < NBLK))
        def _():
            pltpu.make_async_copy(
                tok_hbm.at[pl.ds(base, TB)], tokbuf.at[0], tsem).wait()

        for s in range(DEPTH):
            rr = r + s
            nxt = rr + DEPTH - 1
            nslot = (s + DEPTH - 1) % DEPTH

            wait_gathers(s, gsems[s])

            if s == 0:
                issue_gathers(nxt, (nxt // TB) & 1, nslot, gsems[nslot])
            else:
                @pl.when(g < NG - 1)
                def _():
                    issue_gathers(nxt, (nxt // TB) & 1, nslot, gsems[nslot])

            compute_row(rr, tslot, s)
        return ()

    lax.fori_loop(0, NG, group, ())
    pltpu.sync_copy(outbuf, out_hbm.at[pl.ds(base, RPW)])


_sc_kernel = functools.partial(
    pl.kernel,
    out_type=jax.ShapeDtypeStruct((BATCH, D), jnp.float32),
    mesh=plsc.VectorSubcoreMesh(
        core_axis_name="c", subcore_axis_name="s",
        num_cores=NC, num_subcores=NS),
    scratch_types=[
        pltpu.VMEM((2, TB, HIST), jnp.int32),
        pltpu.VMEM((DEPTH, HIST, D), jnp.float32),
        pltpu.VMEM((RPW, D), jnp.float32),
        pltpu.SemaphoreType.DMA,
        pltpu.SemaphoreType.DMA,
        pltpu.SemaphoreType.DMA,
        pltpu.SemaphoreType.DMA,
        pltpu.SemaphoreType.DMA,
    ],
    compiler_params=pltpu.CompilerParams(
        needs_layout_passes=False, use_tc_tiling_on_sc=False),
)(_sc_body)


def kernel(token_ids, table):
    return _sc_kernel(token_ids, table)
